# baseline (speedup 1.0000x reference)
import functools

import jax
import jax.numpy as jnp
from jax import lax
from jax.experimental import pallas as pl
from jax.experimental.pallas import tpu as pltpu
from jax.experimental.pallas import tpu_sc as plsc

B, S, D, E, TOP_K = 4, 4096, 2048, 16, 2
N_TOK = B * S
T = 1024

NC, NS, L = 2, 16, 16
NW = NC * NS
TOK_PER_W = N_TOK // NW
GROUPS = TOK_PER_W // L
CHUNKS_PER_B = S // TOK_PER_W


def _scores_body(x_ref, w_ref, b_ref, out_ref):
    logits = lax.dot_general(
        w_ref[...], x_ref[...],
        dimension_numbers=(((1,), (1,)), ((), ())),
        preferred_element_type=jnp.float32,
    )
    logits = logits + b_ref[...]
    m = jnp.max(logits, axis=0, keepdims=True)
    e = jnp.exp(logits - m)
    out_ref[...] = (e / jnp.sum(e, axis=0, keepdims=True))[None]


def _dispatch_body(scores_hbm, out_hbm, buf_in, buf_out):
    wid = lax.axis_index("s") * NC + lax.axis_index("c")
    b = wid // CHUNKS_PER_B
    chunk = (wid % CHUNKS_PER_B) * TOK_PER_W
    pltpu.sync_copy(scores_hbm.at[b, :, pl.ds(chunk, TOK_PER_W)], buf_in)

    def group(g):
        gbase = g * L
        vals = [buf_in[e, pl.ds(gbase, L)] for e in range(E)]
        m1 = vals[0]
        i1 = jnp.zeros((L,), jnp.int32)
        m2 = jnp.full((L,), -jnp.inf, jnp.float32)
        i2 = jnp.zeros((L,), jnp.int32)
        for e in range(1, E):
            v = vals[e]
            ev = jnp.full((L,), e, jnp.int32)
            gt1 = v > m1
            gt2 = v > m2
            m2 = jnp.where(gt1, m1, jnp.where(gt2, v, m2))
            i2 = jnp.where(gt1, i1, jnp.where(gt2, ev, i2))
            m1 = jnp.where(gt1, v, m1)
            i1 = jnp.where(gt1, ev, i1)
        for e in range(E):
            ev = jnp.full((L,), e, jnp.int32)
            keep = (i1 == ev) | (i2 == ev)
            buf_out[e, pl.ds(gbase, L)] = jnp.where(keep, vals[e], 0.0)

    plsc.parallel_loop(0, GROUPS, 1, unroll=2)(group)
    pltpu.sync_copy(buf_out, out_hbm.at[b, :, pl.ds(chunk, TOK_PER_W)])


@jax.jit
def kernel(x, W, b):
    xf = x.reshape(N_TOK, D)
    b2 = b.reshape(E, 1)
    scores_t = pl.pallas_call(
        _scores_body,
        grid=(N_TOK // T,),
        in_specs=[
            pl.BlockSpec((T, D), lambda i: (i, 0)),
            pl.BlockSpec((E, D), lambda i: (0, 0)),
            pl.BlockSpec((E, 1), lambda i: (0, 0)),
        ],
        out_specs=pl.BlockSpec((1, E, T), lambda i: (i // (S // T), 0, i % (S // T))),
        out_shape=jax.ShapeDtypeStruct((B, E, S), jnp.float32),
    )(xf, W, b2)

    mesh = plsc.VectorSubcoreMesh(
        core_axis_name="c", subcore_axis_name="s", num_cores=NC
    )
    dispatch_t = pl.kernel(
        _dispatch_body,
        out_type=jax.ShapeDtypeStruct((B, E, S), jnp.float32),
        mesh=mesh,
        compiler_params=pltpu.CompilerParams(
            needs_layout_passes=False, skip_device_barrier=True
        ),
        scratch_types=[
            pltpu.VMEM((E, TOK_PER_W), jnp.float32),
            pltpu.VMEM((E, TOK_PER_W), jnp.float32),
        ],
    )(scores_t)
    return dispatch_t.transpose(0, 2, 1)


# speedup vs baseline: 1.0124x; 1.0124x over previous
import jax
import jax.numpy as jnp
from jax import lax
from jax.experimental import pallas as pl
from jax.experimental.pallas import tpu as pltpu
from jax.experimental.pallas import tpu_sc as plsc

B, S, D, E, TOP_K = 4, 4096, 2048, 16, 2
N_TOK = B * S
T = 1024

NC, NS, L = 2, 16, 16
NW = NC * NS
TOK_PER_W = N_TOK // NW
GROUPS = TOK_PER_W // L
CHUNKS_PER_B = S // TOK_PER_W


def _scores_body(x_ref, w_ref, b_ref, out_ref):
    logits = lax.dot_general(
        w_ref[...], x_ref[...],
        dimension_numbers=(((1,), (1,)), ((), ())),
        preferred_element_type=jnp.float32,
    )
    logits = logits + b_ref[...]
    m = jnp.max(logits, axis=0, keepdims=True)
    e = jnp.exp(logits - m)
    out_ref[...] = (e / jnp.sum(e, axis=0, keepdims=True))[None]


def _dispatch_body(scores_hbm, out_hbm, buf_in, buf_out):
    wid = lax.axis_index("s") * NC + lax.axis_index("c")
    b = wid // CHUNKS_PER_B
    chunk = (wid % CHUNKS_PER_B) * TOK_PER_W
    pltpu.sync_copy(scores_hbm.at[b, :, pl.ds(chunk, TOK_PER_W)], buf_in)

    def group(g, carry):
        gbase = g * L
        vals = [buf_in[e, pl.ds(gbase, L)] for e in range(E)]
        m1 = vals[0]
        i1 = jnp.zeros((L,), jnp.int32)
        m2 = jnp.full((L,), -jnp.inf, jnp.float32)
        i2 = jnp.zeros((L,), jnp.int32)
        for e in range(1, E):
            v = vals[e]
            ev = jnp.full((L,), e, jnp.int32)
            gt1 = v > m1
            gt2 = v > m2
            m2 = jnp.where(gt1, m1, jnp.where(gt2, v, m2))
            i2 = jnp.where(gt1, i1, jnp.where(gt2, ev, i2))
            m1 = jnp.where(gt1, v, m1)
            i1 = jnp.where(gt1, ev, i1)
        for e in range(E):
            ev = jnp.full((L,), e, jnp.int32)
            keep = (i1 == ev) | (i2 == ev)
            buf_out[e, pl.ds(gbase, L)] = jnp.where(keep, vals[e], 0.0)
        return carry

    lax.fori_loop(0, GROUPS, group, 0)
    pltpu.sync_copy(buf_out, out_hbm.at[b, :, pl.ds(chunk, TOK_PER_W)])


@jax.jit
def kernel(x, W, b):
    xf = x.reshape(N_TOK, D)
    b2 = b.reshape(E, 1)
    scores_t = pl.pallas_call(
        _scores_body,
        grid=(N_TOK // T,),
        in_specs=[
            pl.BlockSpec((T, D), lambda i: (i, 0)),
            pl.BlockSpec((E, D), lambda i: (0, 0)),
            pl.BlockSpec((E, 1), lambda i: (0, 0)),
        ],
        out_specs=pl.BlockSpec((1, E, T), lambda i: (i // (S // T), 0, i % (S // T))),
        out_shape=jax.ShapeDtypeStruct((B, E, S), jnp.float32),
    )(xf, W, b2)

    mesh = plsc.VectorSubcoreMesh(
        core_axis_name="c", subcore_axis_name="s", num_cores=NC
    )
    dispatch_t = pl.kernel(
        _dispatch_body,
        out_type=jax.ShapeDtypeStruct((B, E, S), jnp.float32),
        mesh=mesh,
        compiler_params=pltpu.CompilerParams(needs_layout_passes=False),
        scratch_types=[
            pltpu.VMEM((E, TOK_PER_W), jnp.float32),
            pltpu.VMEM((E, TOK_PER_W), jnp.float32),
        ],
    )(scores_t)
    return dispatch_t.transpose(0, 2, 1)
